# baseline (speedup 1.0000x reference)
import functools

import jax
import jax.numpy as jnp
from jax import lax
from jax.experimental import pallas as pl
from jax.experimental.pallas import tpu as pltpu
from jax.experimental.pallas import tpu_sc as plsc

N_NODES = 10000
N_EDGES = 320000
HIDDEN = 128
A = 128

_INFO = plsc.get_sparse_core_info()
_NC = _INFO.num_cores
_NS = _INFO.num_subcores
_NW = _NC * _NS
_EPW = N_EDGES // _NW
_C = 80
_NCHUNK = _EPW // _C
_RPT = 624
_REM_BASE = _NS * _RPT


def _init_shared(zeros_h, sh, sid):
    base = pl.multiple_of(sid * _RPT, 8)
    pltpu.sync_copy(zeros_h.at[pl.ds(base, _RPT)], sh.at[pl.ds(base, _RPT)])

    @pl.when(sid < 2)
    def _():
        off = pl.multiple_of(_REM_BASE + sid * 8, 8)
        pltpu.sync_copy(zeros_h.at[pl.ds(off, 8)], sh.at[pl.ds(off, 8)])


def _copy_out_partial(sh, out_h, cid, sid):
    base = pl.multiple_of(sid * _RPT, 8)
    pltpu.sync_copy(sh.at[pl.ds(base, _RPT)], out_h.at[cid, pl.ds(base, _RPT)])

    @pl.when(sid < 2)
    def _():
        off = pl.multiple_of(_REM_BASE + sid * 8, 8)
        pltpu.sync_copy(sh.at[pl.ds(off, 8)], out_h.at[cid, pl.ds(off, 8)])


def _sc_qk_gather(qh, kh, dst3, src3, n_edges, epw, csz):
    nchunk = epw // csz
    mesh = plsc.VectorSubcoreMesh(core_axis_name="c", subcore_axis_name="s")

    @functools.partial(
        pl.kernel,
        out_type=(
            jax.ShapeDtypeStruct((n_edges, HIDDEN), jnp.float32),
            jax.ShapeDtypeStruct((n_edges, HIDDEN), jnp.float32),
        ),
        mesh=mesh,
        scratch_types=(
            pltpu.VMEM((nchunk, csz), jnp.int32),
            pltpu.VMEM((nchunk, csz), jnp.int32),
            pltpu.VMEM((2, csz, HIDDEN), jnp.float32),
            pltpu.VMEM((2, csz, HIDDEN), jnp.float32),
            pltpu.SemaphoreType.DMA,
            pltpu.SemaphoreType.DMA,
            pltpu.SemaphoreType.DMA,
            pltpu.SemaphoreType.DMA,
        ),
    )
    def k(qh_h, kh_h, dst_h, src_h, qg_h, kg_h,
          dst_v, src_v, qv, kv, sem_g0, sem_g1, sem_w0, sem_w1):
        cid = lax.axis_index("c")
        sid = lax.axis_index("s")
        wid = sid * _NC + cid
        sem_g = (sem_g0, sem_g1)
        sem_w = (sem_w0, sem_w1)

        pltpu.sync_copy(dst_h.at[wid], dst_v)
        pltpu.sync_copy(src_h.at[wid], src_v)

        def issue_gathers(c, b):
            pltpu.async_copy(qh_h.at[dst_v.at[c]], qv.at[b], sem_g[b])
            pltpu.async_copy(kh_h.at[src_v.at[c]], kv.at[b], sem_g[b])

        def wait_gathers(c, b):
            pltpu.make_async_copy(qh_h.at[dst_v.at[c]], qv.at[b], sem_g[b]).wait()
            pltpu.make_async_copy(kh_h.at[src_v.at[c]], kv.at[b], sem_g[b]).wait()

        def issue_writes(c, b):
            base = pl.multiple_of(wid * epw + c * csz, csz)
            pltpu.async_copy(qv.at[b], qg_h.at[pl.ds(base, csz)], sem_w[b])
            pltpu.async_copy(kv.at[b], kg_h.at[pl.ds(base, csz)], sem_w[b])

        def wait_writes(c, b):
            base = pl.multiple_of(wid * epw + c * csz, csz)
            pltpu.make_async_copy(qv.at[b], qg_h.at[pl.ds(base, csz)], sem_w[b]).wait()
            pltpu.make_async_copy(kv.at[b], kg_h.at[pl.ds(base, csz)], sem_w[b]).wait()

        issue_gathers(0, 0)
        wait_gathers(0, 0)
        issue_writes(0, 0)
        issue_gathers(1, 1)
        wait_gathers(1, 1)
        issue_writes(1, 1)
        wait_writes(0, 0)
        issue_gathers(2, 0)

        def body(j, carry):
            for k_ in range(2):
                c = 2 * j + k_
                b = k_
                wait_gathers(c, b)
                issue_writes(c, b)
                wait_writes(c - 1, 1 - b)
                issue_gathers(c + 1, 1 - b)
            return carry

        lax.fori_loop(1, nchunk // 2, body, 0)
        c_last = nchunk - 1
        wait_gathers(c_last, 0)
        issue_writes(c_last, 0)
        wait_writes(c_last - 1, 1)
        wait_writes(c_last, 0)

    return k(qh, kh, dst3, src3)


def _sc_v_segsum(vh, dst3, src3, zeros):
    mesh = plsc.VectorSubcoreMesh(core_axis_name="c", subcore_axis_name="s")

    @functools.partial(
        pl.kernel,
        out_type=jax.ShapeDtypeStruct((_NC, N_NODES, HIDDEN), jnp.float32),
        mesh=mesh,
        scratch_types=(
            pltpu.VMEM((_NCHUNK, _C), jnp.int32),
            pltpu.VMEM((2, _C), jnp.int32),
            pltpu.VMEM((2, _C, HIDDEN), jnp.float32),
            pltpu.VMEM_SHARED((N_NODES, HIDDEN), jnp.float32),
            pltpu.SemaphoreType.DMA,
            pltpu.SemaphoreType.DMA,
            pltpu.SemaphoreType.DMA,
            pltpu.SemaphoreType.DMA,
        ),
    )
    def k(vh_h, dst_h, src_h, zeros_h, aggp_h,
          dst_v, src_v, vv, agg_sh, sem_g0, sem_g1, sem_i0, sem_i1):
        cid = lax.axis_index("c")
        sid = lax.axis_index("s")
        wid = sid * _NC + cid
        sem_g = (sem_g0, sem_g1)
        sem_i = (sem_i0, sem_i1)

        _init_shared(zeros_h, agg_sh, sid)
        pltpu.sync_copy(dst_h.at[wid], dst_v)
        plsc.subcore_barrier()

        def issue_idx(c, b):
            pltpu.async_copy(src_h.at[wid, c], src_v.at[b], sem_i[b])

        def wait_idx(c, b):
            pltpu.make_async_copy(src_h.at[wid, c], src_v.at[b], sem_i[b]).wait()

        def issue_gather(b):
            pltpu.async_copy(vh_h.at[src_v.at[b]], vv.at[b], sem_g[b])

        def wait_gather(b):
            pltpu.make_async_copy(vh_h.at[src_v.at[b]], vv.at[b], sem_g[b]).wait()

        issue_idx(0, 0)
        wait_idx(0, 0)
        issue_gather(0)
        issue_idx(1, 1)

        def body(j, carry):
            for k_ in range(2):
                c = 2 * j + k_
                b = k_
                wait_gather(b)
                wait_idx(c + 1, 1 - b)
                issue_gather(1 - b)

                @pl.when(c + 2 < _NCHUNK)
                def _():
                    issue_idx(c + 2, b)

                pltpu.sync_copy(vv.at[b], agg_sh.at[dst_v.at[c]], add=True)
            return carry

        lax.fori_loop(0, (_NCHUNK - 1) // 2, body, 0)
        c_last = _NCHUNK - 1
        wait_gather(0)
        pltpu.sync_copy(vv.at[0], agg_sh.at[dst_v.at[c_last]], add=True)

        plsc.subcore_barrier()
        _copy_out_partial(agg_sh, aggp_h, cid, sid)

    return k(vh, dst3, src3, zeros)


def _sc_segsum(conn, dst3, zeros, eoff, epw, csz):
    nchunk = epw // csz
    mesh = plsc.VectorSubcoreMesh(core_axis_name="c", subcore_axis_name="s")

    @functools.partial(
        pl.kernel,
        out_type=jax.ShapeDtypeStruct((_NC, N_NODES, HIDDEN), jnp.float32),
        mesh=mesh,
        scratch_types=(
            pltpu.VMEM((nchunk, csz), jnp.int32),
            pltpu.VMEM((2, csz, HIDDEN), jnp.float32),
            pltpu.VMEM_SHARED((N_NODES, HIDDEN), jnp.float32),
            pltpu.SemaphoreType.DMA,
            pltpu.SemaphoreType.DMA,
        ),
    )
    def k(conn_h, dst_h, zeros_h, eaggp_h, dst_v, cv, e_sh, sem_l0, sem_l1):
        cid = lax.axis_index("c")
        sid = lax.axis_index("s")
        wid = sid * _NC + cid
        sem_l = (sem_l0, sem_l1)

        _init_shared(zeros_h, e_sh, sid)
        pltpu.sync_copy(dst_h.at[wid], dst_v)
        plsc.subcore_barrier()

        def issue_load(c, b):
            base = pl.multiple_of(eoff + wid * epw + c * csz, csz)
            pltpu.async_copy(conn_h.at[pl.ds(base, csz)], cv.at[b], sem_l[b])

        def wait_load(c, b):
            base = pl.multiple_of(eoff + wid * epw + c * csz, csz)
            pltpu.make_async_copy(conn_h.at[pl.ds(base, csz)], cv.at[b],
                                  sem_l[b]).wait()

        issue_load(0, 0)

        def body(j, carry):
            for k_ in range(2):
                c = 2 * j + k_
                b = k_
                wait_load(c, b)
                issue_load(c + 1, 1 - b)
                pltpu.sync_copy(cv.at[b], e_sh.at[dst_v.at[c]], add=True)
            return carry

        lax.fori_loop(0, (nchunk - 1) // 2, body, 0)
        c_last = nchunk - 1
        wait_load(c_last, 0)
        pltpu.sync_copy(cv.at[0], e_sh.at[dst_v.at[c_last]], add=True)

        plsc.subcore_barrier()
        _copy_out_partial(e_sh, eaggp_h, cid, sid)

    return k(conn, dst3, zeros)


def _tc_qkv(x, qkv_weight, qkv_bias):
    bm = 1000

    def body(x_ref, w_ref, b_ref, q_ref, k_ref, v_ref):
        r = lax.dot_general(
            x_ref[...], w_ref[...], (((1,), (1,)), ((), ())),
            preferred_element_type=jnp.float32,
            precision=lax.Precision.HIGHEST,
        ) + b_ref[...]
        q_ref[...] = r[:, :A]
        k_ref[...] = r[:, A:2 * A]
        v_ref[...] = r[:, 2 * A:]

    return pl.pallas_call(
        body,
        grid=(N_NODES // bm,),
        in_specs=[
            pl.BlockSpec((bm, HIDDEN), lambda i: (i, 0)),
            pl.BlockSpec((3 * A, HIDDEN), lambda i: (0, 0)),
            pl.BlockSpec((1, 3 * A), lambda i: (0, 0)),
        ],
        out_specs=[pl.BlockSpec((bm, HIDDEN), lambda i: (i, 0))] * 3,
        out_shape=[jax.ShapeDtypeStruct((N_NODES, HIDDEN), jnp.float32)] * 3,
    )(x, qkv_weight, qkv_bias)


_BE = 2000
_HALF = N_EDGES // 2
_HBLK = _HALF // _BE


def _edge_body(pc_ref, qg_ref, kg_ref, w_ref, b_ref, conn_ref):
    eh = lax.dot_general(
        pc_ref[...], w_ref[...], (((1,), (1,)), ((), ())),
        preferred_element_type=jnp.float32,
        precision=lax.Precision.DEFAULT,
    ) + b_ref[...]
    m = qg_ref[...] + kg_ref[...]
    c1 = m * eh[:, :A]
    c2 = jnp.sqrt(jnp.maximum(c1, 0.0)) - jnp.sqrt(jnp.maximum(-c1, 0.0))
    conn_ref[...] = jnp.maximum(c2 + eh[:, A:], 0.0)


def _tc_edge_h0(poly_conn, qg, kg, E_weight, E_bias):
    return pl.pallas_call(
        _edge_body,
        grid=(_HBLK,),
        in_specs=[
            pl.BlockSpec((_BE, HIDDEN), lambda i: (i, 0)),
            pl.BlockSpec((_BE, HIDDEN), lambda i: (i, 0)),
            pl.BlockSpec((_BE, HIDDEN), lambda i: (i, 0)),
            pl.BlockSpec((2 * A, HIDDEN), lambda i: (0, 0)),
            pl.BlockSpec((1, 2 * A), lambda i: (0, 0)),
        ],
        out_specs=pl.BlockSpec((_BE, HIDDEN), lambda i: (i, 0)),
        out_shape=jax.ShapeDtypeStruct((N_EDGES, HIDDEN), jnp.float32),
    )(poly_conn, qg, kg, E_weight, E_bias)


def _tc_edge_h1(conn_prev, poly_conn, qg, kg, E_weight, E_bias):
    def body(prev_ref, pc_ref, qg_ref, kg_ref, w_ref, b_ref, conn_ref):
        del prev_ref
        _edge_body(pc_ref, qg_ref, kg_ref, w_ref, b_ref, conn_ref)

    return pl.pallas_call(
        body,
        grid=(_HBLK,),
        in_specs=[
            pl.BlockSpec(memory_space=pltpu.MemorySpace.HBM),
            pl.BlockSpec((_BE, HIDDEN), lambda i: (i + _HBLK, 0)),
            pl.BlockSpec((_BE, HIDDEN), lambda i: (i, 0)),
            pl.BlockSpec((_BE, HIDDEN), lambda i: (i, 0)),
            pl.BlockSpec((2 * A, HIDDEN), lambda i: (0, 0)),
            pl.BlockSpec((1, 2 * A), lambda i: (0, 0)),
        ],
        out_specs=pl.BlockSpec((_BE, HIDDEN), lambda i: (i + _HBLK, 0)),
        out_shape=jax.ShapeDtypeStruct((N_EDGES, HIDDEN), jnp.float32),
        input_output_aliases={0: 0},
    )(conn_prev, poly_conn, qg, kg, E_weight, E_bias)


def _tc_final(aggp, eaggp0, eaggp1, w, b):
    bm = 1000

    def body(ap_ref, e0_ref, e1_ref, w_ref, b_ref, o_ref):
        eagg = (e0_ref[0] + e0_ref[1]) + (e1_ref[0] + e1_ref[1])
        o_ref[...] = ap_ref[0] + ap_ref[1] + lax.dot_general(
            eagg, w_ref[...], (((1,), (1,)), ((), ())),
            preferred_element_type=jnp.float32,
            precision=lax.Precision.HIGHEST,
        ) + b_ref[...]

    blk3 = pl.BlockSpec((2, bm, HIDDEN), lambda i: (0, i, 0))
    return pl.pallas_call(
        body,
        grid=(N_NODES // bm,),
        in_specs=[blk3, blk3, blk3,
                  pl.BlockSpec((HIDDEN, A), lambda i: (0, 0)),
                  pl.BlockSpec((1, HIDDEN), lambda i: (0, 0))],
        out_specs=pl.BlockSpec((bm, HIDDEN), lambda i: (i, 0)),
        out_shape=jax.ShapeDtypeStruct((N_NODES, HIDDEN), jnp.float32),
    )(aggp, eaggp0, eaggp1, w, b)


_CH = 40
_EPWH = _HALF // _NW


def kernel(x, poly_conn, poly_index, qkv_weight, qkv_bias, E_weight, E_bias,
           conn_lin_weight, conn_lin_bias):
    qh, kh, vh = _tc_qkv(x, qkv_weight, qkv_bias.reshape(1, -1))
    dst = poly_index[0]
    src = poly_index[1]
    dst3 = dst.reshape(_NW, _NCHUNK, _C)
    src3 = src.reshape(_NW, _NCHUNK, _C)
    dst3h = [dst[h * _HALF:(h + 1) * _HALF].reshape(_NW, _EPWH // _CH, _CH)
             for h in range(2)]
    src3h = [src[h * _HALF:(h + 1) * _HALF].reshape(_NW, _EPWH // _CH, _CH)
             for h in range(2)]
    zeros = jnp.zeros((N_NODES, HIDDEN), jnp.float32)
    eb2 = E_bias.reshape(1, -1)

    qg0, kg0 = _sc_qk_gather(qh, kh, dst3h[0], src3h[0], _HALF, _EPWH, _CH)
    qg1, kg1 = _sc_qk_gather(qh, kh, dst3h[1], src3h[1], _HALF, _EPWH, _CH)
    aggp = _sc_v_segsum(vh, dst3, src3, zeros)
    conn_p = _tc_edge_h0(poly_conn, qg0, kg0, E_weight, eb2)
    conn = _tc_edge_h1(conn_p, poly_conn, qg1, kg1, E_weight, eb2)
    eaggp0 = _sc_segsum(conn, dst3h[0], zeros, 0, _EPWH, _CH)
    eaggp1 = _sc_segsum(conn, dst3h[1], zeros, _HALF, _EPWH, _CH)
    no = _tc_final(aggp, eaggp0, eaggp1,
                   conn_lin_weight, conn_lin_bias.reshape(1, -1))
    return no, conn


# speedup vs baseline: 1.2056x; 1.2056x over previous
import functools

import jax
import jax.numpy as jnp
from jax import lax
from jax.experimental import pallas as pl
from jax.experimental.pallas import tpu as pltpu
from jax.experimental.pallas import tpu_sc as plsc

N_NODES = 10000
N_EDGES = 320000
HIDDEN = 128
A = 128

_INFO = plsc.get_sparse_core_info()
_NC = _INFO.num_cores
_NS = _INFO.num_subcores
_NW = _NC * _NS
_EPW = N_EDGES // _NW
_C = 80
_NCHUNK = _EPW // _C
_RPT = 624
_REM_BASE = _NS * _RPT


def _init_shared(zeros_h, sh, sid):
    base = pl.multiple_of(sid * _RPT, 8)
    pltpu.sync_copy(zeros_h.at[pl.ds(base, _RPT)], sh.at[pl.ds(base, _RPT)])

    @pl.when(sid < 2)
    def _():
        off = pl.multiple_of(_REM_BASE + sid * 8, 8)
        pltpu.sync_copy(zeros_h.at[pl.ds(off, 8)], sh.at[pl.ds(off, 8)])


def _copy_out_partial(sh, out_h, cid, sid):
    base = pl.multiple_of(sid * _RPT, 8)
    pltpu.sync_copy(sh.at[pl.ds(base, _RPT)], out_h.at[cid, pl.ds(base, _RPT)])

    @pl.when(sid < 2)
    def _():
        off = pl.multiple_of(_REM_BASE + sid * 8, 8)
        pltpu.sync_copy(sh.at[pl.ds(off, 8)], out_h.at[cid, pl.ds(off, 8)])


def _sc_qk_gather(qh, kh, dst3, src3, n_edges, epw, csz):
    nchunk = epw // csz
    mesh = plsc.VectorSubcoreMesh(core_axis_name="c", subcore_axis_name="s")

    @functools.partial(
        pl.kernel,
        out_type=(
            jax.ShapeDtypeStruct((n_edges, HIDDEN), jnp.float32),
            jax.ShapeDtypeStruct((n_edges, HIDDEN), jnp.float32),
        ),
        mesh=mesh,
        scratch_types=(
            pltpu.VMEM((nchunk, csz), jnp.int32),
            pltpu.VMEM((nchunk, csz), jnp.int32),
            pltpu.VMEM((2, csz, HIDDEN), jnp.float32),
            pltpu.VMEM((2, csz, HIDDEN), jnp.float32),
            pltpu.SemaphoreType.DMA,
            pltpu.SemaphoreType.DMA,
            pltpu.SemaphoreType.DMA,
            pltpu.SemaphoreType.DMA,
        ),
    )
    def k(qh_h, kh_h, dst_h, src_h, qg_h, kg_h,
          dst_v, src_v, qv, kv, sem_g0, sem_g1, sem_w0, sem_w1):
        cid = lax.axis_index("c")
        sid = lax.axis_index("s")
        wid = sid * _NC + cid
        sem_g = (sem_g0, sem_g1)
        sem_w = (sem_w0, sem_w1)

        pltpu.sync_copy(dst_h.at[wid], dst_v)
        pltpu.sync_copy(src_h.at[wid], src_v)

        def issue_gathers(c, b):
            pltpu.async_copy(qh_h.at[dst_v.at[c]], qv.at[b], sem_g[b])
            pltpu.async_copy(kh_h.at[src_v.at[c]], kv.at[b], sem_g[b])

        def wait_gathers(c, b):
            pltpu.make_async_copy(qh_h.at[dst_v.at[c]], qv.at[b], sem_g[b]).wait()
            pltpu.make_async_copy(kh_h.at[src_v.at[c]], kv.at[b], sem_g[b]).wait()

        def issue_writes(c, b):
            base = pl.multiple_of(wid * epw + c * csz, csz)
            pltpu.async_copy(qv.at[b], qg_h.at[pl.ds(base, csz)], sem_w[b])
            pltpu.async_copy(kv.at[b], kg_h.at[pl.ds(base, csz)], sem_w[b])

        def wait_writes(c, b):
            base = pl.multiple_of(wid * epw + c * csz, csz)
            pltpu.make_async_copy(qv.at[b], qg_h.at[pl.ds(base, csz)], sem_w[b]).wait()
            pltpu.make_async_copy(kv.at[b], kg_h.at[pl.ds(base, csz)], sem_w[b]).wait()

        def step(c, b):
            wait_gathers(c, b)
            issue_writes(c, b)

            @pl.when(c >= 1)
            def _():
                wait_writes(c - 1, 1 - b)

            @pl.when(c + 1 < nchunk)
            def _():
                issue_gathers(c + 1, 1 - b)

        issue_gathers(0, 0)

        def body(j, carry):
            for k_ in range(2):
                step(2 * j + k_, k_)
            return carry

        lax.fori_loop(0, nchunk // 2, body, 0)
        if nchunk % 2:
            step(nchunk - 1, 0)
            wait_writes(nchunk - 1, 0)
        else:
            wait_writes(nchunk - 1, 1)

    return k(qh, kh, dst3, src3)


def _sc_v_segsum(vh, dst3, src3, zeros):
    mesh = plsc.VectorSubcoreMesh(core_axis_name="c", subcore_axis_name="s")

    @functools.partial(
        pl.kernel,
        out_type=jax.ShapeDtypeStruct((_NC, N_NODES, HIDDEN), jnp.float32),
        mesh=mesh,
        scratch_types=(
            pltpu.VMEM((_NCHUNK, _C), jnp.int32),
            pltpu.VMEM((2, _C), jnp.int32),
            pltpu.VMEM((2, _C, HIDDEN), jnp.float32),
            pltpu.VMEM_SHARED((N_NODES, HIDDEN), jnp.float32),
            pltpu.SemaphoreType.DMA,
            pltpu.SemaphoreType.DMA,
            pltpu.SemaphoreType.DMA,
            pltpu.SemaphoreType.DMA,
        ),
    )
    def k(vh_h, dst_h, src_h, zeros_h, aggp_h,
          dst_v, src_v, vv, agg_sh, sem_g0, sem_g1, sem_i0, sem_i1):
        cid = lax.axis_index("c")
        sid = lax.axis_index("s")
        wid = sid * _NC + cid
        sem_g = (sem_g0, sem_g1)
        sem_i = (sem_i0, sem_i1)

        _init_shared(zeros_h, agg_sh, sid)
        pltpu.sync_copy(dst_h.at[wid], dst_v)
        plsc.subcore_barrier()

        def issue_idx(c, b):
            pltpu.async_copy(src_h.at[wid, c], src_v.at[b], sem_i[b])

        def wait_idx(c, b):
            pltpu.make_async_copy(src_h.at[wid, c], src_v.at[b], sem_i[b]).wait()

        def issue_gather(b):
            pltpu.async_copy(vh_h.at[src_v.at[b]], vv.at[b], sem_g[b])

        def wait_gather(b):
            pltpu.make_async_copy(vh_h.at[src_v.at[b]], vv.at[b], sem_g[b]).wait()

        issue_idx(0, 0)
        wait_idx(0, 0)
        issue_gather(0)
        issue_idx(1, 1)

        def body(j, carry):
            for k_ in range(2):
                c = 2 * j + k_
                b = k_
                wait_gather(b)
                wait_idx(c + 1, 1 - b)
                issue_gather(1 - b)

                @pl.when(c + 2 < _NCHUNK)
                def _():
                    issue_idx(c + 2, b)

                pltpu.sync_copy(vv.at[b], agg_sh.at[dst_v.at[c]], add=True)
            return carry

        lax.fori_loop(0, (_NCHUNK - 1) // 2, body, 0)
        c_last = _NCHUNK - 1
        wait_gather(0)
        pltpu.sync_copy(vv.at[0], agg_sh.at[dst_v.at[c_last]], add=True)

        plsc.subcore_barrier()
        _copy_out_partial(agg_sh, aggp_h, cid, sid)

    return k(vh, dst3, src3, zeros)


def _sc_segsum(conn, dst3, zeros, eoff, epw, csz):
    nchunk = epw // csz
    mesh = plsc.VectorSubcoreMesh(core_axis_name="c", subcore_axis_name="s")

    @functools.partial(
        pl.kernel,
        out_type=jax.ShapeDtypeStruct((_NC, N_NODES, HIDDEN), jnp.float32),
        mesh=mesh,
        scratch_types=(
            pltpu.VMEM((nchunk, csz), jnp.int32),
            pltpu.VMEM((2, csz, HIDDEN), jnp.float32),
            pltpu.VMEM_SHARED((N_NODES, HIDDEN), jnp.float32),
            pltpu.SemaphoreType.DMA,
            pltpu.SemaphoreType.DMA,
        ),
    )
    def k(conn_h, dst_h, zeros_h, eaggp_h, dst_v, cv, e_sh, sem_l0, sem_l1):
        cid = lax.axis_index("c")
        sid = lax.axis_index("s")
        wid = sid * _NC + cid
        sem_l = (sem_l0, sem_l1)

        _init_shared(zeros_h, e_sh, sid)
        pltpu.sync_copy(dst_h.at[wid], dst_v)
        plsc.subcore_barrier()

        def issue_load(c, b):
            base = pl.multiple_of(eoff + wid * epw + c * csz, csz)
            pltpu.async_copy(conn_h.at[pl.ds(base, csz)], cv.at[b], sem_l[b])

        def wait_load(c, b):
            base = pl.multiple_of(eoff + wid * epw + c * csz, csz)
            pltpu.make_async_copy(conn_h.at[pl.ds(base, csz)], cv.at[b],
                                  sem_l[b]).wait()

        def step(c, b):
            wait_load(c, b)

            @pl.when(c + 1 < nchunk)
            def _():
                issue_load(c + 1, 1 - b)

            pltpu.sync_copy(cv.at[b], e_sh.at[dst_v.at[c]], add=True)

        issue_load(0, 0)

        def body(j, carry):
            for k_ in range(2):
                step(2 * j + k_, k_)
            return carry

        lax.fori_loop(0, nchunk // 2, body, 0)
        if nchunk % 2:
            step(nchunk - 1, 0)

        plsc.subcore_barrier()
        _copy_out_partial(e_sh, eaggp_h, cid, sid)

    return k(conn, dst3, zeros)


def _tc_qkv(x, qkv_weight, qkv_bias):
    bm = 1000

    def body(x_ref, w_ref, b_ref, q_ref, k_ref, v_ref):
        r = lax.dot_general(
            x_ref[...], w_ref[...], (((1,), (1,)), ((), ())),
            preferred_element_type=jnp.float32,
            precision=lax.Precision.HIGHEST,
        ) + b_ref[...]
        q_ref[...] = r[:, :A]
        k_ref[...] = r[:, A:2 * A]
        v_ref[...] = r[:, 2 * A:]

    return pl.pallas_call(
        body,
        grid=(N_NODES // bm,),
        in_specs=[
            pl.BlockSpec((bm, HIDDEN), lambda i: (i, 0)),
            pl.BlockSpec((3 * A, HIDDEN), lambda i: (0, 0)),
            pl.BlockSpec((1, 3 * A), lambda i: (0, 0)),
        ],
        out_specs=[pl.BlockSpec((bm, HIDDEN), lambda i: (i, 0))] * 3,
        out_shape=[jax.ShapeDtypeStruct((N_NODES, HIDDEN), jnp.float32)] * 3,
    )(x, qkv_weight, qkv_bias)


_BE = 2000
_E0 = 192000
_E1 = N_EDGES - _E0


def _edge_body(pc_ref, qg_ref, kg_ref, w_ref, b_ref, conn_ref):
    eh = lax.dot_general(
        pc_ref[...], w_ref[...], (((1,), (1,)), ((), ())),
        preferred_element_type=jnp.float32,
        precision=lax.Precision.DEFAULT,
    ) + b_ref[...]
    m = qg_ref[...] + kg_ref[...]
    c1 = m * eh[:, :A]
    c2 = jnp.sqrt(jnp.maximum(c1, 0.0)) - jnp.sqrt(jnp.maximum(-c1, 0.0))
    conn_ref[...] = jnp.maximum(c2 + eh[:, A:], 0.0)


def _tc_edge_h0(poly_conn, qg, kg, E_weight, E_bias):
    return pl.pallas_call(
        _edge_body,
        grid=(_E0 // _BE,),
        in_specs=[
            pl.BlockSpec((_BE, HIDDEN), lambda i: (i, 0)),
            pl.BlockSpec((_BE, HIDDEN), lambda i: (i, 0)),
            pl.BlockSpec((_BE, HIDDEN), lambda i: (i, 0)),
            pl.BlockSpec((2 * A, HIDDEN), lambda i: (0, 0)),
            pl.BlockSpec((1, 2 * A), lambda i: (0, 0)),
        ],
        out_specs=pl.BlockSpec((_BE, HIDDEN), lambda i: (i, 0)),
        out_shape=jax.ShapeDtypeStruct((N_EDGES, HIDDEN), jnp.float32),
    )(poly_conn, qg, kg, E_weight, E_bias)


def _tc_edge_h1(conn_prev, poly_conn, qg, kg, E_weight, E_bias):
    off = _E0 // _BE

    def body(prev_ref, pc_ref, qg_ref, kg_ref, w_ref, b_ref, conn_ref):
        del prev_ref
        _edge_body(pc_ref, qg_ref, kg_ref, w_ref, b_ref, conn_ref)

    return pl.pallas_call(
        body,
        grid=(_E1 // _BE,),
        in_specs=[
            pl.BlockSpec(memory_space=pltpu.MemorySpace.HBM),
            pl.BlockSpec((_BE, HIDDEN), lambda i: (i + off, 0)),
            pl.BlockSpec((_BE, HIDDEN), lambda i: (i, 0)),
            pl.BlockSpec((_BE, HIDDEN), lambda i: (i, 0)),
            pl.BlockSpec((2 * A, HIDDEN), lambda i: (0, 0)),
            pl.BlockSpec((1, 2 * A), lambda i: (0, 0)),
        ],
        out_specs=pl.BlockSpec((_BE, HIDDEN), lambda i: (i + off, 0)),
        out_shape=jax.ShapeDtypeStruct((N_EDGES, HIDDEN), jnp.float32),
        input_output_aliases={0: 0},
    )(conn_prev, poly_conn, qg, kg, E_weight, E_bias)


def _tc_final(aggp, eaggp0, eaggp1, w, b):
    bm = 1000

    def body(ap_ref, e0_ref, e1_ref, w_ref, b_ref, o_ref):
        eagg = (e0_ref[0] + e0_ref[1]) + (e1_ref[0] + e1_ref[1])
        o_ref[...] = ap_ref[0] + ap_ref[1] + lax.dot_general(
            eagg, w_ref[...], (((1,), (1,)), ((), ())),
            preferred_element_type=jnp.float32,
            precision=lax.Precision.HIGHEST,
        ) + b_ref[...]

    blk3 = pl.BlockSpec((2, bm, HIDDEN), lambda i: (0, i, 0))
    return pl.pallas_call(
        body,
        grid=(N_NODES // bm,),
        in_specs=[blk3, blk3, blk3,
                  pl.BlockSpec((HIDDEN, A), lambda i: (0, 0)),
                  pl.BlockSpec((1, HIDDEN), lambda i: (0, 0))],
        out_specs=pl.BlockSpec((bm, HIDDEN), lambda i: (i, 0)),
        out_shape=jax.ShapeDtypeStruct((N_NODES, HIDDEN), jnp.float32),
    )(aggp, eaggp0, eaggp1, w, b)


_EPW0 = _E0 // _NW
_EPW1 = _E1 // _NW


def kernel(x, poly_conn, poly_index, qkv_weight, qkv_bias, E_weight, E_bias,
           conn_lin_weight, conn_lin_bias):
    qh, kh, vh = _tc_qkv(x, qkv_weight, qkv_bias.reshape(1, -1))
    dst = poly_index[0]
    src = poly_index[1]
    dst3 = dst.reshape(_NW, _NCHUNK, _C)
    src3 = src.reshape(_NW, _NCHUNK, _C)
    dst3s = [dst[:_E0].reshape(_NW, _EPW0 // _C, _C),
             dst[_E0:].reshape(_NW, _EPW1 // _C, _C)]
    src3s = [src[:_E0].reshape(_NW, _EPW0 // _C, _C),
             src[_E0:].reshape(_NW, _EPW1 // _C, _C)]
    zeros = jnp.zeros((N_NODES, HIDDEN), jnp.float32)
    eb2 = E_bias.reshape(1, -1)

    qg0, kg0 = _sc_qk_gather(qh, kh, dst3s[0], src3s[0], _E0, _EPW0, _C)
    qg1, kg1 = _sc_qk_gather(qh, kh, dst3s[1], src3s[1], _E1, _EPW1, _C)
    aggp = _sc_v_segsum(vh, dst3, src3, zeros)
    conn_p = _tc_edge_h0(poly_conn, qg0, kg0, E_weight, eb2)
    conn = _tc_edge_h1(conn_p, poly_conn, qg1, kg1, E_weight, eb2)
    eaggp0 = _sc_segsum(conn, dst3s[0], zeros, 0, _EPW0, _C)
    eaggp1 = _sc_segsum(conn, dst3s[1], zeros, _E0, _EPW1, _C)
    no = _tc_final(aggp, eaggp0, eaggp1,
                   conn_lin_weight, conn_lin_bias.reshape(1, -1))
    return no, conn


# speedup vs baseline: 1.3390x; 1.1106x over previous
import functools

import jax
import jax.numpy as jnp
from jax import lax
from jax.experimental import pallas as pl
from jax.experimental.pallas import tpu as pltpu
from jax.experimental.pallas import tpu_sc as plsc

N_NODES = 10000
N_EDGES = 320000
HIDDEN = 128
A = 128

_INFO = plsc.get_sparse_core_info()
_NC = _INFO.num_cores
_NS = _INFO.num_subcores
_NW = _NC * _NS
_EPW = N_EDGES // _NW
_C = 80
_NCHUNK = _EPW // _C
_RPT = 624
_REM_BASE = _NS * _RPT


def _init_shared(zeros_h, sh, sid):
    base = pl.multiple_of(sid * _RPT, 8)
    pltpu.sync_copy(zeros_h.at[pl.ds(base, _RPT)], sh.at[pl.ds(base, _RPT)])

    @pl.when(sid < 2)
    def _():
        off = pl.multiple_of(_REM_BASE + sid * 8, 8)
        pltpu.sync_copy(zeros_h.at[pl.ds(off, 8)], sh.at[pl.ds(off, 8)])


def _copy_out_partial(sh, out_h, cid, sid):
    base = pl.multiple_of(sid * _RPT, 8)
    pltpu.sync_copy(sh.at[pl.ds(base, _RPT)], out_h.at[cid, pl.ds(base, _RPT)])

    @pl.when(sid < 2)
    def _():
        off = pl.multiple_of(_REM_BASE + sid * 8, 8)
        pltpu.sync_copy(sh.at[pl.ds(off, 8)], out_h.at[cid, pl.ds(off, 8)])


def _sc_g_gather(qh, kh, dst3, src3, n_edges, epw, csz):
    nchunk = epw // csz
    mesh = plsc.VectorSubcoreMesh(core_axis_name="c", subcore_axis_name="s")

    @functools.partial(
        pl.kernel,
        out_type=jax.ShapeDtypeStruct((n_edges, HIDDEN), jnp.float32),
        mesh=mesh,
        scratch_types=(
            pltpu.VMEM((nchunk, csz), jnp.int32),
            pltpu.VMEM((nchunk, csz), jnp.int32),
            pltpu.VMEM((2, csz, HIDDEN), jnp.float32),
            pltpu.SemaphoreType.DMA,
            pltpu.SemaphoreType.DMA,
            pltpu.SemaphoreType.DMA,
            pltpu.SemaphoreType.DMA,
            pltpu.SemaphoreType.DMA,
            pltpu.SemaphoreType.DMA,
        ),
    )
    def k(qh_h, kh_h, dst_h, src_h, g_h,
          dst_v, src_v, gv, sem_q0, sem_q1, sem_a0, sem_a1, sem_w0, sem_w1):
        cid = lax.axis_index("c")
        sid = lax.axis_index("s")
        wid = sid * _NC + cid
        sem_q = (sem_q0, sem_q1)
        sem_a = (sem_a0, sem_a1)
        sem_w = (sem_w0, sem_w1)

        pltpu.sync_copy(dst_h.at[wid], dst_v)
        pltpu.sync_copy(src_h.at[wid], src_v)

        def issue_q(c, b):
            pltpu.async_copy(qh_h.at[dst_v.at[c]], gv.at[b], sem_q[b])

        def wait_q(c, b):
            pltpu.make_async_copy(qh_h.at[dst_v.at[c]], gv.at[b], sem_q[b]).wait()

        def issue_kadd(c, b):
            pltpu.async_copy(kh_h.at[src_v.at[c]], gv.at[b], sem_a[b], add=True)

        def wait_kadd(c, b):
            pltpu.make_async_copy(kh_h.at[src_v.at[c]], gv.at[b], sem_a[b]).wait()

        def issue_write(c, b):
            base = pl.multiple_of(wid * epw + c * csz, csz)
            pltpu.async_copy(gv.at[b], g_h.at[pl.ds(base, csz)], sem_w[b])

        def wait_write(c, b):
            base = pl.multiple_of(wid * epw + c * csz, csz)
            pltpu.make_async_copy(gv.at[b], g_h.at[pl.ds(base, csz)], sem_w[b]).wait()

        def step(c, b):
            wait_q(c, b)
            issue_kadd(c, b)

            @pl.when(c >= 1)
            def _():
                wait_write(c - 1, 1 - b)

            @pl.when(c + 1 < nchunk)
            def _():
                issue_q(c + 1, 1 - b)

            wait_kadd(c, b)
            issue_write(c, b)

        issue_q(0, 0)

        def body(j, carry):
            for k_ in range(2):
                step(2 * j + k_, k_)
            return carry

        lax.fori_loop(0, nchunk // 2, body, 0)
        if nchunk % 2:
            step(nchunk - 1, 0)
            wait_write(nchunk - 1, 0)
        else:
            wait_write(nchunk - 1, 1)

    return k(qh, kh, dst3, src3)


def _sc_v_segsum(vh, dst3, src3, zeros):
    mesh = plsc.VectorSubcoreMesh(core_axis_name="c", subcore_axis_name="s")

    @functools.partial(
        pl.kernel,
        out_type=jax.ShapeDtypeStruct((_NC, N_NODES, HIDDEN), jnp.float32),
        mesh=mesh,
        scratch_types=(
            pltpu.VMEM((_NCHUNK, _C), jnp.int32),
            pltpu.VMEM((2, _C), jnp.int32),
            pltpu.VMEM((2, _C, HIDDEN), jnp.float32),
            pltpu.VMEM_SHARED((N_NODES, HIDDEN), jnp.float32),
            pltpu.SemaphoreType.DMA,
            pltpu.SemaphoreType.DMA,
            pltpu.SemaphoreType.DMA,
            pltpu.SemaphoreType.DMA,
        ),
    )
    def k(vh_h, dst_h, src_h, zeros_h, aggp_h,
          dst_v, src_v, vv, agg_sh, sem_g0, sem_g1, sem_i0, sem_i1):
        cid = lax.axis_index("c")
        sid = lax.axis_index("s")
        wid = sid * _NC + cid
        sem_g = (sem_g0, sem_g1)
        sem_i = (sem_i0, sem_i1)

        _init_shared(zeros_h, agg_sh, sid)
        pltpu.sync_copy(dst_h.at[wid], dst_v)
        plsc.subcore_barrier()

        def issue_idx(c, b):
            pltpu.async_copy(src_h.at[wid, c], src_v.at[b], sem_i[b])

        def wait_idx(c, b):
            pltpu.make_async_copy(src_h.at[wid, c], src_v.at[b], sem_i[b]).wait()

        def issue_gather(b):
            pltpu.async_copy(vh_h.at[src_v.at[b]], vv.at[b], sem_g[b])

        def wait_gather(b):
            pltpu.make_async_copy(vh_h.at[src_v.at[b]], vv.at[b], sem_g[b]).wait()

        issue_idx(0, 0)
        wait_idx(0, 0)
        issue_gather(0)
        issue_idx(1, 1)

        def body(j, carry):
            for k_ in range(2):
                c = 2 * j + k_
                b = k_
                wait_gather(b)
                wait_idx(c + 1, 1 - b)
                issue_gather(1 - b)

                @pl.when(c + 2 < _NCHUNK)
                def _():
                    issue_idx(c + 2, b)

                pltpu.sync_copy(vv.at[b], agg_sh.at[dst_v.at[c]], add=True)
            return carry

        lax.fori_loop(0, (_NCHUNK - 1) // 2, body, 0)
        c_last = _NCHUNK - 1
        wait_gather(0)
        pltpu.sync_copy(vv.at[0], agg_sh.at[dst_v.at[c_last]], add=True)

        plsc.subcore_barrier()
        _copy_out_partial(agg_sh, aggp_h, cid, sid)

    return k(vh, dst3, src3, zeros)


def _sc_segsum(conn, dst3, zeros, eoff, epw, csz):
    nchunk = epw // csz
    mesh = plsc.VectorSubcoreMesh(core_axis_name="c", subcore_axis_name="s")

    @functools.partial(
        pl.kernel,
        out_type=jax.ShapeDtypeStruct((_NC, N_NODES, HIDDEN), jnp.float32),
        mesh=mesh,
        scratch_types=(
            pltpu.VMEM((nchunk, csz), jnp.int32),
            pltpu.VMEM((2, csz, HIDDEN), jnp.float32),
            pltpu.VMEM_SHARED((N_NODES, HIDDEN), jnp.float32),
            pltpu.SemaphoreType.DMA,
            pltpu.SemaphoreType.DMA,
        ),
    )
    def k(conn_h, dst_h, zeros_h, eaggp_h, dst_v, cv, e_sh, sem_l0, sem_l1):
        cid = lax.axis_index("c")
        sid = lax.axis_index("s")
        wid = sid * _NC + cid
        sem_l = (sem_l0, sem_l1)

        _init_shared(zeros_h, e_sh, sid)
        pltpu.sync_copy(dst_h.at[wid], dst_v)
        plsc.subcore_barrier()

        def issue_load(c, b):
            base = pl.multiple_of(eoff + wid * epw + c * csz, csz)
            pltpu.async_copy(conn_h.at[pl.ds(base, csz)], cv.at[b], sem_l[b])

        def wait_load(c, b):
            base = pl.multiple_of(eoff + wid * epw + c * csz, csz)
            pltpu.make_async_copy(conn_h.at[pl.ds(base, csz)], cv.at[b],
                                  sem_l[b]).wait()

        def step(c, b):
            wait_load(c, b)

            @pl.when(c + 1 < nchunk)
            def _():
                issue_load(c + 1, 1 - b)

            pltpu.sync_copy(cv.at[b], e_sh.at[dst_v.at[c]], add=True)

        issue_load(0, 0)

        def body(j, carry):
            for k_ in range(2):
                step(2 * j + k_, k_)
            return carry

        lax.fori_loop(0, nchunk // 2, body, 0)
        if nchunk % 2:
            step(nchunk - 1, 0)

        plsc.subcore_barrier()
        _copy_out_partial(e_sh, eaggp_h, cid, sid)

    return k(conn, dst3, zeros)


def _tc_qkv(x, qkv_weight, qkv_bias):
    bm = 1000

    def body(x_ref, w_ref, b_ref, q_ref, k_ref, v_ref):
        r = lax.dot_general(
            x_ref[...], w_ref[...], (((1,), (1,)), ((), ())),
            preferred_element_type=jnp.float32,
            precision=lax.Precision.HIGHEST,
        ) + b_ref[...]
        q_ref[...] = r[:, :A]
        k_ref[...] = r[:, A:2 * A]
        v_ref[...] = r[:, 2 * A:]

    return pl.pallas_call(
        body,
        grid=(N_NODES // bm,),
        in_specs=[
            pl.BlockSpec((bm, HIDDEN), lambda i: (i, 0)),
            pl.BlockSpec((3 * A, HIDDEN), lambda i: (0, 0)),
            pl.BlockSpec((1, 3 * A), lambda i: (0, 0)),
        ],
        out_specs=[pl.BlockSpec((bm, HIDDEN), lambda i: (i, 0))] * 3,
        out_shape=[jax.ShapeDtypeStruct((N_NODES, HIDDEN), jnp.float32)] * 3,
    )(x, qkv_weight, qkv_bias)


_BE = 2000
_E0 = 192000
_E1 = N_EDGES - _E0


def _edge_body(pc_ref, g_ref, w_ref, b_ref, conn_ref):
    eh = lax.dot_general(
        pc_ref[...], w_ref[...], (((1,), (1,)), ((), ())),
        preferred_element_type=jnp.float32,
        precision=lax.Precision.DEFAULT,
    ) + b_ref[...]
    c1 = g_ref[...] * eh[:, :A]
    c2 = jnp.sqrt(jnp.maximum(c1, 0.0)) - jnp.sqrt(jnp.maximum(-c1, 0.0))
    conn_ref[...] = jnp.maximum(c2 + eh[:, A:], 0.0)


def _tc_edge_h0(poly_conn, g, E_weight, E_bias):
    return pl.pallas_call(
        _edge_body,
        grid=(_E0 // _BE,),
        in_specs=[
            pl.BlockSpec((_BE, HIDDEN), lambda i: (i, 0)),
            pl.BlockSpec((_BE, HIDDEN), lambda i: (i, 0)),
            pl.BlockSpec((2 * A, HIDDEN), lambda i: (0, 0)),
            pl.BlockSpec((1, 2 * A), lambda i: (0, 0)),
        ],
        out_specs=pl.BlockSpec((_BE, HIDDEN), lambda i: (i, 0)),
        out_shape=jax.ShapeDtypeStruct((N_EDGES, HIDDEN), jnp.float32),
    )(poly_conn, g, E_weight, E_bias)


def _tc_edge_h1(conn_prev, poly_conn, g, E_weight, E_bias):
    off = _E0 // _BE

    def body(prev_ref, pc_ref, g_ref, w_ref, b_ref, conn_ref):
        del prev_ref
        _edge_body(pc_ref, g_ref, w_ref, b_ref, conn_ref)

    return pl.pallas_call(
        body,
        grid=(_E1 // _BE,),
        in_specs=[
            pl.BlockSpec(memory_space=pltpu.MemorySpace.HBM),
            pl.BlockSpec((_BE, HIDDEN), lambda i: (i + off, 0)),
            pl.BlockSpec((_BE, HIDDEN), lambda i: (i, 0)),
            pl.BlockSpec((2 * A, HIDDEN), lambda i: (0, 0)),
            pl.BlockSpec((1, 2 * A), lambda i: (0, 0)),
        ],
        out_specs=pl.BlockSpec((_BE, HIDDEN), lambda i: (i + off, 0)),
        out_shape=jax.ShapeDtypeStruct((N_EDGES, HIDDEN), jnp.float32),
        input_output_aliases={0: 0},
    )(conn_prev, poly_conn, g, E_weight, E_bias)


def _tc_final(aggp, eaggp0, eaggp1, w, b):
    bm = 1000

    def body(ap_ref, e0_ref, e1_ref, w_ref, b_ref, o_ref):
        eagg = (e0_ref[0] + e0_ref[1]) + (e1_ref[0] + e1_ref[1])
        o_ref[...] = ap_ref[0] + ap_ref[1] + lax.dot_general(
            eagg, w_ref[...], (((1,), (1,)), ((), ())),
            preferred_element_type=jnp.float32,
            precision=lax.Precision.HIGHEST,
        ) + b_ref[...]

    blk3 = pl.BlockSpec((2, bm, HIDDEN), lambda i: (0, i, 0))
    return pl.pallas_call(
        body,
        grid=(N_NODES // bm,),
        in_specs=[blk3, blk3, blk3,
                  pl.BlockSpec((HIDDEN, A), lambda i: (0, 0)),
                  pl.BlockSpec((1, HIDDEN), lambda i: (0, 0))],
        out_specs=pl.BlockSpec((bm, HIDDEN), lambda i: (i, 0)),
        out_shape=jax.ShapeDtypeStruct((N_NODES, HIDDEN), jnp.float32),
    )(aggp, eaggp0, eaggp1, w, b)


_EPW0 = _E0 // _NW
_EPW1 = _E1 // _NW


def kernel(x, poly_conn, poly_index, qkv_weight, qkv_bias, E_weight, E_bias,
           conn_lin_weight, conn_lin_bias):
    qh, kh, vh = _tc_qkv(x, qkv_weight, qkv_bias.reshape(1, -1))
    dst = poly_index[0]
    src = poly_index[1]
    dst3 = dst.reshape(_NW, _NCHUNK, _C)
    src3 = src.reshape(_NW, _NCHUNK, _C)
    dst3s = [dst[:_E0].reshape(_NW, _EPW0 // _C, _C),
             dst[_E0:].reshape(_NW, _EPW1 // _C, _C)]
    src3s = [src[:_E0].reshape(_NW, _EPW0 // _C, _C),
             src[_E0:].reshape(_NW, _EPW1 // _C, _C)]
    zeros = jnp.zeros((N_NODES, HIDDEN), jnp.float32)
    eb2 = E_bias.reshape(1, -1)

    g0 = _sc_g_gather(qh, kh, dst3s[0], src3s[0], _E0, _EPW0, _C)
    g1 = _sc_g_gather(qh, kh, dst3s[1], src3s[1], _E1, _EPW1, _C)
    aggp = _sc_v_segsum(vh, dst3, src3, zeros)
    conn_p = _tc_edge_h0(poly_conn, g0, E_weight, eb2)
    conn = _tc_edge_h1(conn_p, poly_conn, g1, E_weight, eb2)
    eaggp0 = _sc_segsum(conn, dst3s[0], zeros, 0, _EPW0, _C)
    eaggp1 = _sc_segsum(conn, dst3s[1], zeros, _E0, _EPW1, _C)
    no = _tc_final(aggp, eaggp0, eaggp1,
                   conn_lin_weight, conn_lin_bias.reshape(1, -1))
    return no, conn


# speedup vs baseline: 1.3422x; 1.0024x over previous
import functools

import jax
import jax.numpy as jnp
from jax import lax
from jax.experimental import pallas as pl
from jax.experimental.pallas import tpu as pltpu
from jax.experimental.pallas import tpu_sc as plsc

N_NODES = 10000
N_EDGES = 320000
HIDDEN = 128
A = 128

_INFO = plsc.get_sparse_core_info()
_NC = _INFO.num_cores
_NS = _INFO.num_subcores
_NW = _NC * _NS
_EPW = N_EDGES // _NW
_C = 80
_NCHUNK = _EPW // _C
_RPT = 624
_REM_BASE = _NS * _RPT


def _init_shared(zeros_h, sh, sid):
    base = pl.multiple_of(sid * _RPT, 8)
    pltpu.sync_copy(zeros_h.at[pl.ds(base, _RPT)], sh.at[pl.ds(base, _RPT)])

    @pl.when(sid < 2)
    def _():
        off = pl.multiple_of(_REM_BASE + sid * 8, 8)
        pltpu.sync_copy(zeros_h.at[pl.ds(off, 8)], sh.at[pl.ds(off, 8)])


def _copy_out_partial(sh, out_h, cid, sid):
    base = pl.multiple_of(sid * _RPT, 8)
    pltpu.sync_copy(sh.at[pl.ds(base, _RPT)], out_h.at[cid, pl.ds(base, _RPT)])

    @pl.when(sid < 2)
    def _():
        off = pl.multiple_of(_REM_BASE + sid * 8, 8)
        pltpu.sync_copy(sh.at[pl.ds(off, 8)], out_h.at[cid, pl.ds(off, 8)])


def _sc_g_gather(qh, kh, dst3, src3, n_edges, epw, csz):
    nchunk = epw // csz
    mesh = plsc.VectorSubcoreMesh(core_axis_name="c", subcore_axis_name="s")

    @functools.partial(
        pl.kernel,
        out_type=jax.ShapeDtypeStruct((n_edges, HIDDEN), jnp.float32),
        mesh=mesh,
        scratch_types=(
            pltpu.VMEM((nchunk, csz), jnp.int32),
            pltpu.VMEM((nchunk, csz), jnp.int32),
            pltpu.VMEM((2, csz, HIDDEN), jnp.float32),
            pltpu.SemaphoreType.DMA,
            pltpu.SemaphoreType.DMA,
            pltpu.SemaphoreType.DMA,
            pltpu.SemaphoreType.DMA,
            pltpu.SemaphoreType.DMA,
            pltpu.SemaphoreType.DMA,
        ),
    )
    def k(qh_h, kh_h, dst_h, src_h, g_h,
          dst_v, src_v, gv, sem_q0, sem_q1, sem_a0, sem_a1, sem_w0, sem_w1):
        cid = lax.axis_index("c")
        sid = lax.axis_index("s")
        wid = sid * _NC + cid
        sem_q = (sem_q0, sem_q1)
        sem_a = (sem_a0, sem_a1)
        sem_w = (sem_w0, sem_w1)

        pltpu.sync_copy(dst_h.at[wid], dst_v)
        pltpu.sync_copy(src_h.at[wid], src_v)

        def issue_q(c, b):
            pltpu.async_copy(qh_h.at[dst_v.at[c]], gv.at[b], sem_q[b])

        def wait_q(c, b):
            pltpu.make_async_copy(qh_h.at[dst_v.at[c]], gv.at[b], sem_q[b]).wait()

        def issue_kadd(c, b):
            pltpu.async_copy(kh_h.at[src_v.at[c]], gv.at[b], sem_a[b], add=True)

        def wait_kadd(c, b):
            pltpu.make_async_copy(kh_h.at[src_v.at[c]], gv.at[b], sem_a[b]).wait()

        def issue_write(c, b):
            base = pl.multiple_of(wid * epw + c * csz, csz)
            pltpu.async_copy(gv.at[b], g_h.at[pl.ds(base, csz)], sem_w[b])

        def wait_write(c, b):
            base = pl.multiple_of(wid * epw + c * csz, csz)
            pltpu.make_async_copy(gv.at[b], g_h.at[pl.ds(base, csz)], sem_w[b]).wait()

        def step(c, b):
            wait_q(c, b)
            issue_kadd(c, b)

            @pl.when(c >= 1)
            def _():
                wait_write(c - 1, 1 - b)

            @pl.when(c + 1 < nchunk)
            def _():
                issue_q(c + 1, 1 - b)

            wait_kadd(c, b)
            issue_write(c, b)

        issue_q(0, 0)

        def body(j, carry):
            for k_ in range(2):
                step(2 * j + k_, k_)
            return carry

        lax.fori_loop(0, nchunk // 2, body, 0)
        if nchunk % 2:
            step(nchunk - 1, 0)
            wait_write(nchunk - 1, 0)
        else:
            wait_write(nchunk - 1, 1)

    return k(qh, kh, dst3, src3)


def _sc_v_segsum(vh, dst3, src3, zeros):
    mesh = plsc.VectorSubcoreMesh(core_axis_name="c", subcore_axis_name="s")

    @functools.partial(
        pl.kernel,
        out_type=jax.ShapeDtypeStruct((_NC, N_NODES, HIDDEN), jnp.float32),
        mesh=mesh,
        scratch_types=(
            pltpu.VMEM((_NCHUNK, _C), jnp.int32),
            pltpu.VMEM((2, _C), jnp.int32),
            pltpu.VMEM((2, _C, HIDDEN), jnp.float32),
            pltpu.VMEM_SHARED((N_NODES, HIDDEN), jnp.float32),
            pltpu.SemaphoreType.DMA,
            pltpu.SemaphoreType.DMA,
            pltpu.SemaphoreType.DMA,
            pltpu.SemaphoreType.DMA,
            pltpu.SemaphoreType.DMA,
            pltpu.SemaphoreType.DMA,
        ),
    )
    def k(vh_h, dst_h, src_h, zeros_h, aggp_h,
          dst_v, src_v, vv, agg_sh,
          sem_g0, sem_g1, sem_i0, sem_i1, sem_s0, sem_s1):
        cid = lax.axis_index("c")
        sid = lax.axis_index("s")
        wid = sid * _NC + cid
        sem_g = (sem_g0, sem_g1)
        sem_i = (sem_i0, sem_i1)
        sem_s = (sem_s0, sem_s1)

        _init_shared(zeros_h, agg_sh, sid)
        pltpu.sync_copy(dst_h.at[wid], dst_v)
        plsc.subcore_barrier()

        def issue_idx(c, b):
            pltpu.async_copy(src_h.at[wid, c], src_v.at[b], sem_i[b])

        def wait_idx(c, b):
            pltpu.make_async_copy(src_h.at[wid, c], src_v.at[b], sem_i[b]).wait()

        def issue_gather(b):
            pltpu.async_copy(vh_h.at[src_v.at[b]], vv.at[b], sem_g[b])

        def wait_gather(b):
            pltpu.make_async_copy(vh_h.at[src_v.at[b]], vv.at[b], sem_g[b]).wait()

        def issue_scatter(c, b):
            pltpu.async_copy(vv.at[b], agg_sh.at[dst_v.at[c]], sem_s[b], add=True)

        def wait_scatter(c, b):
            pltpu.make_async_copy(vv.at[b], agg_sh.at[dst_v.at[c]], sem_s[b]).wait()

        issue_idx(0, 0)
        wait_idx(0, 0)
        issue_gather(0)
        issue_idx(1, 1)

        def step(c, b):
            wait_gather(b)
            issue_scatter(c, b)

            @pl.when(c + 1 < _NCHUNK)
            def _():
                wait_idx(c + 1, 1 - b)

                @pl.when(c >= 1)
                def __():
                    wait_scatter(c - 1, 1 - b)

                issue_gather(1 - b)

            @pl.when(c + 2 < _NCHUNK)
            def _():
                issue_idx(c + 2, b)

        def body(j, carry):
            for k_ in range(2):
                step(2 * j + k_, k_)
            return carry

        lax.fori_loop(0, _NCHUNK // 2, body, 0)
        if _NCHUNK % 2:
            step(_NCHUNK - 1, 0)
        wait_scatter(_NCHUNK - 2, (_NCHUNK - 2) % 2)
        wait_scatter(_NCHUNK - 1, (_NCHUNK - 1) % 2)

        plsc.subcore_barrier()
        _copy_out_partial(agg_sh, aggp_h, cid, sid)

    return k(vh, dst3, src3, zeros)


def _sc_segsum(conn, dst3, zeros, eoff, epw, csz):
    nchunk = epw // csz
    mesh = plsc.VectorSubcoreMesh(core_axis_name="c", subcore_axis_name="s")

    @functools.partial(
        pl.kernel,
        out_type=jax.ShapeDtypeStruct((_NC, N_NODES, HIDDEN), jnp.float32),
        mesh=mesh,
        scratch_types=(
            pltpu.VMEM((nchunk, csz), jnp.int32),
            pltpu.VMEM((2, csz, HIDDEN), jnp.float32),
            pltpu.VMEM_SHARED((N_NODES, HIDDEN), jnp.float32),
            pltpu.SemaphoreType.DMA,
            pltpu.SemaphoreType.DMA,
            pltpu.SemaphoreType.DMA,
            pltpu.SemaphoreType.DMA,
        ),
    )
    def k(conn_h, dst_h, zeros_h, eaggp_h, dst_v, cv, e_sh,
          sem_l0, sem_l1, sem_s0, sem_s1):
        cid = lax.axis_index("c")
        sid = lax.axis_index("s")
        wid = sid * _NC + cid
        sem_l = (sem_l0, sem_l1)
        sem_s = (sem_s0, sem_s1)

        _init_shared(zeros_h, e_sh, sid)
        pltpu.sync_copy(dst_h.at[wid], dst_v)
        plsc.subcore_barrier()

        def issue_load(c, b):
            base = pl.multiple_of(eoff + wid * epw + c * csz, csz)
            pltpu.async_copy(conn_h.at[pl.ds(base, csz)], cv.at[b], sem_l[b])

        def wait_load(c, b):
            base = pl.multiple_of(eoff + wid * epw + c * csz, csz)
            pltpu.make_async_copy(conn_h.at[pl.ds(base, csz)], cv.at[b],
                                  sem_l[b]).wait()

        def issue_scatter(c, b):
            pltpu.async_copy(cv.at[b], e_sh.at[dst_v.at[c]], sem_s[b], add=True)

        def wait_scatter(c, b):
            pltpu.make_async_copy(cv.at[b], e_sh.at[dst_v.at[c]], sem_s[b]).wait()

        def step(c, b):
            wait_load(c, b)
            issue_scatter(c, b)

            @pl.when(c + 1 < nchunk)
            def _():
                @pl.when(c >= 1)
                def __():
                    wait_scatter(c - 1, 1 - b)

                issue_load(c + 1, 1 - b)

        issue_load(0, 0)

        def body(j, carry):
            for k_ in range(2):
                step(2 * j + k_, k_)
            return carry

        lax.fori_loop(0, nchunk // 2, body, 0)
        if nchunk % 2:
            step(nchunk - 1, 0)
        wait_scatter(nchunk - 2, (nchunk - 2) % 2)
        wait_scatter(nchunk - 1, (nchunk - 1) % 2)

        plsc.subcore_barrier()
        _copy_out_partial(e_sh, eaggp_h, cid, sid)

    return k(conn, dst3, zeros)


def _tc_qkv(x, qkv_weight, qkv_bias):
    bm = 1000

    def body(x_ref, w_ref, b_ref, q_ref, k_ref, v_ref):
        r = lax.dot_general(
            x_ref[...], w_ref[...], (((1,), (1,)), ((), ())),
            preferred_element_type=jnp.float32,
            precision=lax.Precision.HIGHEST,
        ) + b_ref[...]
        q_ref[...] = r[:, :A]
        k_ref[...] = r[:, A:2 * A]
        v_ref[...] = r[:, 2 * A:]

    return pl.pallas_call(
        body,
        grid=(N_NODES // bm,),
        in_specs=[
            pl.BlockSpec((bm, HIDDEN), lambda i: (i, 0)),
            pl.BlockSpec((3 * A, HIDDEN), lambda i: (0, 0)),
            pl.BlockSpec((1, 3 * A), lambda i: (0, 0)),
        ],
        out_specs=[pl.BlockSpec((bm, HIDDEN), lambda i: (i, 0))] * 3,
        out_shape=[jax.ShapeDtypeStruct((N_NODES, HIDDEN), jnp.float32)] * 3,
    )(x, qkv_weight, qkv_bias)


_BE = 2000
_E0 = 192000
_E1 = N_EDGES - _E0


def _edge_body(pc_ref, g_ref, w_ref, b_ref, conn_ref):
    eh = lax.dot_general(
        pc_ref[...], w_ref[...], (((1,), (1,)), ((), ())),
        preferred_element_type=jnp.float32,
        precision=lax.Precision.DEFAULT,
    ) + b_ref[...]
    c1 = g_ref[...] * eh[:, :A]
    c2 = jnp.sqrt(jnp.maximum(c1, 0.0)) - jnp.sqrt(jnp.maximum(-c1, 0.0))
    conn_ref[...] = jnp.maximum(c2 + eh[:, A:], 0.0)


def _tc_edge_h0(poly_conn, g, E_weight, E_bias):
    return pl.pallas_call(
        _edge_body,
        grid=(_E0 // _BE,),
        in_specs=[
            pl.BlockSpec((_BE, HIDDEN), lambda i: (i, 0)),
            pl.BlockSpec((_BE, HIDDEN), lambda i: (i, 0)),
            pl.BlockSpec((2 * A, HIDDEN), lambda i: (0, 0)),
            pl.BlockSpec((1, 2 * A), lambda i: (0, 0)),
        ],
        out_specs=pl.BlockSpec((_BE, HIDDEN), lambda i: (i, 0)),
        out_shape=jax.ShapeDtypeStruct((N_EDGES, HIDDEN), jnp.float32),
    )(poly_conn, g, E_weight, E_bias)


def _tc_edge_h1(conn_prev, poly_conn, g, E_weight, E_bias):
    off = _E0 // _BE

    def body(prev_ref, pc_ref, g_ref, w_ref, b_ref, conn_ref):
        del prev_ref
        _edge_body(pc_ref, g_ref, w_ref, b_ref, conn_ref)

    return pl.pallas_call(
        body,
        grid=(_E1 // _BE,),
        in_specs=[
            pl.BlockSpec(memory_space=pltpu.MemorySpace.HBM),
            pl.BlockSpec((_BE, HIDDEN), lambda i: (i + off, 0)),
            pl.BlockSpec((_BE, HIDDEN), lambda i: (i, 0)),
            pl.BlockSpec((2 * A, HIDDEN), lambda i: (0, 0)),
            pl.BlockSpec((1, 2 * A), lambda i: (0, 0)),
        ],
        out_specs=pl.BlockSpec((_BE, HIDDEN), lambda i: (i + off, 0)),
        out_shape=jax.ShapeDtypeStruct((N_EDGES, HIDDEN), jnp.float32),
        input_output_aliases={0: 0},
    )(conn_prev, poly_conn, g, E_weight, E_bias)


def _tc_final(aggp, eaggp0, eaggp1, w, b):
    bm = 1000

    def body(ap_ref, e0_ref, e1_ref, w_ref, b_ref, o_ref):
        eagg = (e0_ref[0] + e0_ref[1]) + (e1_ref[0] + e1_ref[1])
        o_ref[...] = ap_ref[0] + ap_ref[1] + lax.dot_general(
            eagg, w_ref[...], (((1,), (1,)), ((), ())),
            preferred_element_type=jnp.float32,
            precision=lax.Precision.HIGHEST,
        ) + b_ref[...]

    blk3 = pl.BlockSpec((2, bm, HIDDEN), lambda i: (0, i, 0))
    return pl.pallas_call(
        body,
        grid=(N_NODES // bm,),
        in_specs=[blk3, blk3, blk3,
                  pl.BlockSpec((HIDDEN, A), lambda i: (0, 0)),
                  pl.BlockSpec((1, HIDDEN), lambda i: (0, 0))],
        out_specs=pl.BlockSpec((bm, HIDDEN), lambda i: (i, 0)),
        out_shape=jax.ShapeDtypeStruct((N_NODES, HIDDEN), jnp.float32),
    )(aggp, eaggp0, eaggp1, w, b)


_EPW0 = _E0 // _NW
_EPW1 = _E1 // _NW


def kernel(x, poly_conn, poly_index, qkv_weight, qkv_bias, E_weight, E_bias,
           conn_lin_weight, conn_lin_bias):
    qh, kh, vh = _tc_qkv(x, qkv_weight, qkv_bias.reshape(1, -1))
    dst = poly_index[0]
    src = poly_index[1]
    dst3 = dst.reshape(_NW, _NCHUNK, _C)
    src3 = src.reshape(_NW, _NCHUNK, _C)
    dst3s = [dst[:_E0].reshape(_NW, _EPW0 // _C, _C),
             dst[_E0:].reshape(_NW, _EPW1 // _C, _C)]
    src3s = [src[:_E0].reshape(_NW, _EPW0 // _C, _C),
             src[_E0:].reshape(_NW, _EPW1 // _C, _C)]
    zeros = jnp.zeros((N_NODES, HIDDEN), jnp.float32)
    eb2 = E_bias.reshape(1, -1)

    g0 = _sc_g_gather(qh, kh, dst3s[0], src3s[0], _E0, _EPW0, _C)
    g1 = _sc_g_gather(qh, kh, dst3s[1], src3s[1], _E1, _EPW1, _C)
    aggp = _sc_v_segsum(vh, dst3, src3, zeros)
    conn_p = _tc_edge_h0(poly_conn, g0, E_weight, eb2)
    conn = _tc_edge_h1(conn_p, poly_conn, g1, E_weight, eb2)
    eaggp0 = _sc_segsum(conn, dst3s[0], zeros, 0, _EPW0, _C)
    eaggp1 = _sc_segsum(conn, dst3s[1], zeros, _E0, _EPW1, _C)
    no = _tc_final(aggp, eaggp0, eaggp1,
                   conn_lin_weight, conn_lin_bias.reshape(1, -1))
    return no, conn
